# Initial kernel scaffold; baseline (speedup 1.0000x reference)
import jax
import jax.numpy as jnp
from jax.experimental import pallas as pl


def kernel(values, tok_table, pos_table):
    raise NotImplementedError("write your pallas kernel here")


# speedup vs baseline: 3.1072x; 3.1072x over previous
import functools

import jax
import jax.numpy as jnp
from jax import lax
from jax.experimental import pallas as pl
from jax.experimental.pallas import tpu as pltpu
from jax.experimental.pallas import tpu_sc as plsc

_VOCAB = 100000
_T = 200
_E = 64
_B = 4096

_NC = 2
_NS = 16
_NW = _NC * _NS
_ROWS_PER_W = _B // _NW
_C0 = 128
_C1 = _T - _C0
_LANES = 16


def _sc_embed(values, tok_table, pos_table):
  mesh = plsc.VectorSubcoreMesh(core_axis_name="c", subcore_axis_name="s")

  @functools.partial(
      pl.kernel,
      mesh=mesh,
      compiler_params=pltpu.CompilerParams(use_tc_tiling_on_sc=False),
      out_type=jax.ShapeDtypeStruct((_B, _T, _E), jnp.float32),
      scratch_types=[
          pltpu.VMEM((_T,), jnp.int32),
          pltpu.VMEM((_T, _E), jnp.float32),
          pltpu.VMEM((_T, _E), jnp.float32),
          pltpu.SemaphoreType.DMA,
      ],
  )
  def k(values_hbm, tok_hbm, pos_hbm, out_hbm, idx_v, rows_v, pos_v, sem):
    wid = lax.axis_index("s") * _NC + lax.axis_index("c")
    pltpu.sync_copy(pos_hbm, pos_v)
    base = wid * _ROWS_PER_W

    def row_body(i, carry):
      b = base + i
      pltpu.sync_copy(values_hbm.at[b], idx_v)
      cp0 = pltpu.async_copy(
          tok_hbm.at[idx_v.at[pl.ds(0, _C0)]], rows_v.at[pl.ds(0, _C0)], sem)
      cp1 = pltpu.async_copy(
          tok_hbm.at[idx_v.at[pl.ds(_C0, _C1)]], rows_v.at[pl.ds(_C0, _C1)], sem)
      cp0.wait()
      cp1.wait()

      def add_body(r, c2):
        for c in range(_E // _LANES):
          sl = pl.ds(c * _LANES, _LANES)
          rows_v[r, sl] = rows_v[r, sl] + pos_v[r, sl]
        return c2

      lax.fori_loop(0, _T, add_body, 0)
      pltpu.sync_copy(rows_v, out_hbm.at[b])
      return carry

    lax.fori_loop(0, _ROWS_PER_W, row_body, 0)

  return k(values, tok_table, pos_table)


def kernel(values, tok_table, pos_table):
  return _sc_embed(values.astype(jnp.int32), tok_table, pos_table)


# speedup vs baseline: 4.1311x; 1.3296x over previous
import functools

import jax
import jax.numpy as jnp
from jax import lax
from jax.experimental import pallas as pl
from jax.experimental.pallas import tpu as pltpu
from jax.experimental.pallas import tpu_sc as plsc

_VOCAB = 100000
_T = 200
_E = 64
_B = 4096

_NC = 2
_NS = 16
_NW = _NC * _NS
_ROWS_PER_W = _B // _NW
_C0 = 128
_C1 = _T - _C0
_LANES = 16
_NBUF = 4


def _sc_embed(values, tok_table, pos_table):
  mesh = plsc.VectorSubcoreMesh(core_axis_name="c", subcore_axis_name="s")

  @functools.partial(
      pl.kernel,
      mesh=mesh,
      compiler_params=pltpu.CompilerParams(use_tc_tiling_on_sc=False),
      out_type=jax.ShapeDtypeStruct((_B, _T, _E), jnp.float32),
      scratch_types=(
          [pltpu.VMEM((_ROWS_PER_W, _T), jnp.int32)]
          + [pltpu.VMEM((_T, _E), jnp.float32) for _ in range(_NBUF + 1)]
          + [pltpu.SemaphoreType.DMA for _ in range(2 * _NBUF)]
      ),
  )
  def k(values_hbm, tok_hbm, pos_hbm, out_hbm, idx_v, *rest):
    bufs = rest[:_NBUF]
    pos_v = rest[_NBUF]
    gsems = rest[_NBUF + 1:_NBUF + 1 + _NBUF]
    ssems = rest[_NBUF + 1 + _NBUF:]

    wid = lax.axis_index("s") * _NC + lax.axis_index("c")
    base = wid * _ROWS_PER_W
    pltpu.sync_copy(pos_hbm, pos_v)
    pltpu.sync_copy(values_hbm.at[pl.ds(base, _ROWS_PER_W)], idx_v)

    def fire_gather(m, i):
      pltpu.async_copy(
          tok_hbm.at[idx_v.at[i, pl.ds(0, _C0)]],
          bufs[m].at[pl.ds(0, _C0)], gsems[m])
      pltpu.async_copy(
          tok_hbm.at[idx_v.at[i, pl.ds(_C0, _C1)]],
          bufs[m].at[pl.ds(_C0, _C1)], gsems[m])

    def wait_gather(m, i):
      pltpu.make_async_copy(
          tok_hbm.at[idx_v.at[i, pl.ds(0, _C0)]],
          bufs[m].at[pl.ds(0, _C0)], gsems[m]).wait()
      pltpu.make_async_copy(
          tok_hbm.at[idx_v.at[i, pl.ds(_C0, _C1)]],
          bufs[m].at[pl.ds(_C0, _C1)], gsems[m]).wait()

    def add_pos(m):
      def add_body(r, c2):
        for c in range(_E // _LANES):
          sl = pl.ds(c * _LANES, _LANES)
          bufs[m][r, sl] = bufs[m][r, sl] + pos_v[r, sl]
        return c2
      lax.fori_loop(0, _T, add_body, 0)

    def wait_store(m, b):
      pltpu.make_async_copy(bufs[m], out_hbm.at[b], ssems[m]).wait()

    fire_gather(0, 0)

    def step(j, carry):
      for kk in range(_NBUF):
        i = j * _NBUF + kk
        nxt = (kk + 1) % _NBUF

        if kk < _NBUF - 1:
          @pl.when(j > 0)
          def _():
            wait_store(nxt, base + (j - 1) * _NBUF + nxt)
          fire_gather(nxt, i + 1)
        else:
          @pl.when(j < _ROWS_PER_W // _NBUF - 1)
          def _():
            wait_store(0, base + j * _NBUF)
            fire_gather(0, i + 1)

        wait_gather(kk, i)
        add_pos(kk)
        pltpu.async_copy(bufs[kk], out_hbm.at[base + i], ssems[kk])
      return carry

    lax.fori_loop(0, _ROWS_PER_W // _NBUF, step, 0)

    for kk in range(_NBUF):
      wait_store(kk, base + _ROWS_PER_W - _NBUF + kk)

  return k(values, tok_table, pos_table)


def kernel(values, tok_table, pos_table):
  return _sc_embed(values.astype(jnp.int32), tok_table, pos_table)
